# Initial kernel scaffold; baseline (speedup 1.0000x reference)
import jax
import jax.numpy as jnp
from jax.experimental import pallas as pl


def kernel(features, W1h, b1h, W2h, b2h, W1o, b1o, W2o, b2o):
    raise NotImplementedError("write your pallas kernel here")


# speedup vs baseline: 1.6162x; 1.6162x over previous
import functools

import jax
import jax.numpy as jnp
from jax import lax
from jax.experimental import pallas as pl

INST = 32
THR = 0.01


def _conv_body(x0, x1, x2, w1, w2, b1, b2, o):
    xs = (x0, x1, x2)
    acc = None
    for dy in range(3):
        for dx in range(3):
            a = xs[dx][0, pl.ds(dy * 64, 4096), :]
            w = w1[pl.ds((dy * 3 + dx) * 256, 256), :]
            t = lax.dot_general(a, w, (((1,), (0,)), ((), ())),
                                preferred_element_type=jnp.float32)
            acc = t if acc is None else acc + t
    r = jnp.maximum(acc + b1[...], 0.0).astype(jnp.bfloat16)
    logits = lax.dot_general(r, w2[...], (((1,), (0,)), ((), ())),
                             preferred_element_type=jnp.float32) + b2[...]
    o[0] = jax.nn.sigmoid(logits)


def _conv_heads(xdx, w1, w2, b1, b2, B):
    return pl.pallas_call(
        _conv_body,
        grid=(B,),
        in_specs=[
            pl.BlockSpec((1, 66 * 64, 256), lambda b: (b, 0, 0)),
            pl.BlockSpec((1, 66 * 64, 256), lambda b: (b, 0, 0)),
            pl.BlockSpec((1, 66 * 64, 256), lambda b: (b, 0, 0)),
            pl.BlockSpec((9 * 256, 512), lambda b: (0, 0)),
            pl.BlockSpec((512, 8), lambda b: (0, 0)),
            pl.BlockSpec((1, 512), lambda b: (0, 0)),
            pl.BlockSpec((1, 8), lambda b: (0, 0)),
        ],
        out_specs=pl.BlockSpec((1, 4096, 8), lambda b: (b, 0, 0)),
        out_shape=jax.ShapeDtypeStruct((B, 4096, 8), jnp.float32),
    )(*xdx, w1, w2, b1, b2)


def kernel(features, W1h, b1h, W2h, b2h, W1o, b1o, W2o, b2o):
    B, C, H, W = features.shape
    HEAD = W1h.shape[0]
    xpad = jnp.pad(features.transpose(0, 2, 3, 1),
                   ((0, 0), (1, 1), (1, 1), (0, 0))).astype(jnp.bfloat16)
    xdx = [xpad[:, :, dx:dx + W, :].reshape(B, (H + 2) * W, C)
           for dx in range(3)]
    w1 = jnp.concatenate([W1h, W1o], axis=0).transpose(2, 3, 1, 0) \
        .reshape(9 * C, 2 * HEAD).astype(jnp.bfloat16)
    w2 = jnp.zeros((2 * HEAD, 8), jnp.float32)
    w2 = w2.at[:HEAD, 0].set(W2h.reshape(HEAD))
    w2 = w2.at[HEAD:, 1].set(W2o.reshape(2, HEAD)[0])
    w2 = w2.at[HEAD:, 2].set(W2o.reshape(2, HEAD)[1]).astype(jnp.bfloat16)
    b1 = jnp.concatenate([b1h, b1o]).reshape(1, 2 * HEAD)
    b2 = jnp.zeros((1, 8), jnp.float32).at[0, 0].set(b2h[0]) \
        .at[0, 1].set(b2o[0]).at[0, 2].set(b2o[1])

    out = _conv_heads(xdx, w1, w2, b1, b2, B)

    pred_hm = out[:, :, 0].reshape(B, 1, H, W)
    pred_offset = out[:, :, 1:3].transpose(0, 2, 1).reshape(B, 2, H, W)

    hmax = lax.reduce_window(
        pred_hm, -jnp.inf, lax.max,
        window_dimensions=(1, 1, 3, 3), window_strides=(1, 1, 1, 1),
        padding=[(0, 0), (0, 0), (1, 1), (1, 1)])
    peak_mask = (pred_hm == hmax) & (pred_hm > THR)
    scores_flat = jnp.where(peak_mask, pred_hm, 0.0).reshape(B, H * W)
    top_scr, top_idx = lax.top_k(scores_flat, INST)
    valid = top_scr > 0.0
    ys = (top_idx // W).astype(jnp.float32)
    xs = (top_idx % W).astype(jnp.float32)
    peak_indices = jnp.where(valid[..., None], jnp.stack([ys, xs], axis=-1), 0.0)
    sampling = (peak_indices[..., 0] * W + peak_indices[..., 1]).astype(jnp.int32)
    off_flat = pred_offset.reshape(B, 2, H * W)
    gather_idx = jnp.broadcast_to(sampling[:, None, :], (B, 2, INST))
    peak_offsets = jnp.transpose(
        jnp.take_along_axis(off_flat, gather_idx, axis=2), (0, 2, 1))
    peak_point = (peak_indices + peak_offsets) / jnp.array(
        [H - 1, W - 1], jnp.float32)
    return pred_hm, pred_offset, top_scr, peak_point
